# baseline (speedup 1.0000x reference)
import jax
import jax.numpy as jnp
from jax import lax
from jax.experimental import pallas as pl

D_MODEL = 768
E = 64


def _router_body(x_ref, w_ref, p1_ref, p2_ref, i1_ref, i2_ref, psum_ref, ent_ref):
    T = x_ref.shape[0]
    logits = lax.dot_general(
        w_ref[:], x_ref[:], (((1,), (1,)), ((), ())),
        preferred_element_type=jnp.float32)
    row = lax.broadcasted_iota(jnp.int32, (E, T), 0)

    m = jnp.max(logits, axis=0, keepdims=True)
    e = jnp.exp(logits - m)
    s = jnp.sum(e, axis=0, keepdims=True)
    r = 1.0 / s
    q = jnp.sum(e * (logits - m), axis=0, keepdims=True)

    i1 = jnp.min(jnp.where(logits == m, row, E), axis=0, keepdims=True)
    lm = jnp.where(row == i1, -jnp.inf, logits)
    m2 = jnp.max(lm, axis=0, keepdims=True)
    i2 = jnp.min(jnp.where(lm == m2, row, E), axis=0, keepdims=True)

    p1_ref[:] = r.reshape(1, 1, T)
    p2_ref[:] = (jnp.exp(m2 - m) / s).reshape(1, 1, T)
    i1_ref[:] = i1.reshape(1, 1, T)
    i2_ref[:] = i2.reshape(1, 1, T)

    @pl.when(pl.program_id(0) == 0)
    def _init():
        psum_ref[:] = jnp.zeros_like(psum_ref)
        ent_ref[:] = jnp.zeros_like(ent_ref)

    psum_ref[:] += jnp.sum(e * r, axis=1, keepdims=True)
    ent_ref[:] += jnp.sum(jnp.log(s) - q * r).reshape(1, 1)


def _mask_body(i1_ref, i2_ref, mask_ref, msum_ref):
    R = mask_ref.shape[0]
    lane = lax.broadcasted_iota(jnp.int32, (R, 2 * E), 1)
    exp_lane = jnp.bitwise_and(lane, E - 1)
    odd = lane >= E
    i1s = jnp.where(odd, i1_ref[:, 1:2], i1_ref[:, 0:1])
    i2s = jnp.where(odd, i2_ref[:, 1:2], i2_ref[:, 0:1])
    mask = ((exp_lane == i1s) | (exp_lane == i2s)).astype(jnp.float32)
    mask_ref[:] = mask

    @pl.when(pl.program_id(0) == 0)
    def _init():
        msum_ref[:] = jnp.zeros_like(msum_ref)

    msum_ref[:] += jnp.sum(mask, axis=0, keepdims=True)


def kernel(x, W, temp):
    B, S, D = x.shape
    N = B * S
    t = jnp.clip(temp, 0.1, 5.0)
    w = W / t
    xf = x.reshape(N, D)
    T = 4096
    grid = N // T

    p1, p2, i1, i2, psum, ent = pl.pallas_call(
        _router_body,
        grid=(grid,),
        in_specs=[
            pl.BlockSpec((T, D), lambda i: (i, 0)),
            pl.BlockSpec((E, D), lambda i: (0, 0)),
        ],
        out_specs=[
            pl.BlockSpec((1, 1, T), lambda i: (i, 0, 0)),
            pl.BlockSpec((1, 1, T), lambda i: (i, 0, 0)),
            pl.BlockSpec((1, 1, T), lambda i: (i, 0, 0)),
            pl.BlockSpec((1, 1, T), lambda i: (i, 0, 0)),
            pl.BlockSpec((E, 1), lambda i: (0, 0)),
            pl.BlockSpec((1, 1), lambda i: (0, 0)),
        ],
        out_shape=[
            jax.ShapeDtypeStruct((grid, 1, T), jnp.float32),
            jax.ShapeDtypeStruct((grid, 1, T), jnp.float32),
            jax.ShapeDtypeStruct((grid, 1, T), jnp.int32),
            jax.ShapeDtypeStruct((grid, 1, T), jnp.int32),
            jax.ShapeDtypeStruct((E, 1), jnp.float32),
            jax.ShapeDtypeStruct((1, 1), jnp.float32),
        ],
    )(xf, w)

    RB = 2048
    mgrid = (N // 2) // RB
    mask2, msum2 = pl.pallas_call(
        _mask_body,
        grid=(mgrid,),
        in_specs=[
            pl.BlockSpec((RB, 2), lambda i: (i, 0)),
            pl.BlockSpec((RB, 2), lambda i: (i, 0)),
        ],
        out_specs=[
            pl.BlockSpec((RB, 2 * E), lambda i: (i, 0)),
            pl.BlockSpec((1, 2 * E), lambda i: (0, 0)),
        ],
        out_shape=[
            jax.ShapeDtypeStruct((N // 2, 2 * E), jnp.float32),
            jax.ShapeDtypeStruct((1, 2 * E), jnp.float32),
        ],
    )(i1.reshape(N // 2, 2), i2.reshape(N // 2, 2))

    expert_probs = jnp.stack([p1.reshape(B, S), p2.reshape(B, S)], axis=-1)
    expert_indices = jnp.stack([i1.reshape(B, S), i2.reshape(B, S)], axis=-1)
    expert_mask = mask2.reshape(B, S, E)

    denom = jnp.float32(N)
    importance = psum[:, 0] / denom
    load = (msum2[0, :E] + msum2[0, E:]) / (denom + 1e-6)
    aux_load_loss = jnp.sum(importance * load) * E * 0.01
    router_entropy = (ent[0, 0] / denom) * 0.01
    return expert_probs, expert_indices, expert_mask, aux_load_loss, router_entropy


# speedup vs baseline: 1.2401x; 1.2401x over previous
import jax
import jax.numpy as jnp
from jax import lax
from jax.experimental import pallas as pl

D_MODEL = 768
E = 64


def _router_body(x_ref, w_ref, p_ref, i_ref, mask_ref, psum_ref, msum_ref, ent_ref):
    T = x_ref.shape[0]
    logits = lax.dot_general(
        w_ref[:], x_ref[:], (((1,), (1,)), ((), ())),
        preferred_element_type=jnp.float32)
    row = lax.broadcasted_iota(jnp.int32, (E, T), 0)

    m = jnp.max(logits, axis=0, keepdims=True)
    e = jnp.exp(logits - m)
    s = jnp.sum(e, axis=0, keepdims=True)
    r = 1.0 / s
    q = jnp.sum(e * (logits - m), axis=0, keepdims=True)

    i1 = jnp.min(jnp.where(logits == m, row, E), axis=0, keepdims=True)
    lm = jnp.where(row == i1, -jnp.inf, logits)
    m2 = jnp.max(lm, axis=0, keepdims=True)
    i2 = jnp.min(jnp.where(lm == m2, row, E), axis=0, keepdims=True)
    p2 = jnp.exp(m2 - m) / s

    p_ref[:] = jnp.transpose(jnp.concatenate([r, p2], axis=0))
    ii = jnp.transpose(jnp.concatenate([i1, i2], axis=0))
    i_ref[:] = ii

    lane = lax.broadcasted_iota(jnp.int32, (T, E), 1)
    mask = ((lane == ii[:, 0:1]) | (lane == ii[:, 1:2])).astype(jnp.float32)
    mask_ref[:] = mask

    @pl.when(pl.program_id(0) == 0)
    def _init():
        psum_ref[:] = jnp.zeros_like(psum_ref)
        msum_ref[:] = jnp.zeros_like(msum_ref)
        ent_ref[:] = jnp.zeros_like(ent_ref)

    psum_ref[:] += jnp.sum(e * r, axis=1, keepdims=True)
    msum_ref[:] += jnp.sum(mask, axis=0, keepdims=True)
    ent_ref[:] += jnp.sum(jnp.log(s) - q * r).reshape(1, 1)


def kernel(x, W, temp):
    B, S, D = x.shape
    N = B * S
    t = jnp.clip(temp, 0.1, 5.0)
    w = W / t
    xf = x.reshape(N, D)
    T = 4096
    grid = N // T

    p_pair, i_pair, mask, psum, msum, ent = pl.pallas_call(
        _router_body,
        grid=(grid,),
        in_specs=[
            pl.BlockSpec((T, D), lambda i: (i, 0)),
            pl.BlockSpec((E, D), lambda i: (0, 0)),
        ],
        out_specs=[
            pl.BlockSpec((T, 2), lambda i: (i, 0)),
            pl.BlockSpec((T, 2), lambda i: (i, 0)),
            pl.BlockSpec((T, E), lambda i: (i, 0)),
            pl.BlockSpec((E, 1), lambda i: (0, 0)),
            pl.BlockSpec((1, E), lambda i: (0, 0)),
            pl.BlockSpec((1, 1), lambda i: (0, 0)),
        ],
        out_shape=[
            jax.ShapeDtypeStruct((N, 2), jnp.float32),
            jax.ShapeDtypeStruct((N, 2), jnp.int32),
            jax.ShapeDtypeStruct((N, E), jnp.float32),
            jax.ShapeDtypeStruct((E, 1), jnp.float32),
            jax.ShapeDtypeStruct((1, E), jnp.float32),
            jax.ShapeDtypeStruct((1, 1), jnp.float32),
        ],
    )(xf, w)

    expert_probs = p_pair.reshape(B, S, 2)
    expert_indices = i_pair.reshape(B, S, 2)
    expert_mask = mask.reshape(B, S, E)

    denom = jnp.float32(N)
    importance = psum[:, 0] / denom
    load = msum[0] / (denom + 1e-6)
    aux_load_loss = jnp.sum(importance * load) * E * 0.01
    router_entropy = (ent[0, 0] / denom) * 0.01
    return expert_probs, expert_indices, expert_mask, aux_load_loss, router_entropy
